# baseline (speedup 1.0000x reference)
import functools

import jax
import jax.numpy as jnp
from jax import lax
from jax.experimental import pallas as pl
from jax.experimental.pallas import tpu as pltpu
from jax.experimental.pallas import tpu_sc as plsc

BATCH = 16384
FEATURES = 128

_info = plsc.get_sparse_core_info()
_NC, _NS = _info.num_cores, _info.num_subcores
_NW = _NC * _NS
_ROWS = BATCH // _NW

_mesh = plsc.VectorSubcoreMesh(core_axis_name="c", subcore_axis_name="s")


@functools.partial(
    pl.kernel,
    mesh=_mesh,
    out_type=jax.ShapeDtypeStruct((BATCH, FEATURES), jnp.float32),
)
def _sc_copy(x_hbm, out_hbm):
    wid = lax.axis_index("s") * _NC + lax.axis_index("c")
    base = wid * _ROWS
    pltpu.sync_copy(x_hbm.at[pl.ds(base, _ROWS)], out_hbm.at[pl.ds(base, _ROWS)])


def kernel(x, queue):
    del queue
    return _sc_copy(x)


# speedup vs baseline: 11.0354x; 11.0354x over previous
import functools

import jax
import jax.numpy as jnp
from jax import lax
from jax.experimental import pallas as pl
from jax.experimental.pallas import tpu as pltpu
from jax.experimental.pallas import tpu_sc as plsc

BATCH = 16384
FEATURES = 128

_info = plsc.get_sparse_core_info()
_NC, _NS = _info.num_cores, _info.num_subcores
_NW = _NC * _NS
_ROWS = BATCH // _NW

_mesh = plsc.VectorSubcoreMesh(core_axis_name="c", subcore_axis_name="s")


@functools.partial(
    pl.kernel,
    mesh=_mesh,
    out_type=jax.ShapeDtypeStruct((BATCH, FEATURES), jnp.float32),
    scratch_types=[pltpu.VMEM((_ROWS, FEATURES), jnp.float32)],
)
def _sc_copy(x_hbm, out_hbm, buf):
    wid = lax.axis_index("s") * _NC + lax.axis_index("c")
    base = wid * _ROWS
    pltpu.sync_copy(x_hbm.at[pl.ds(base, _ROWS)], buf)
    pltpu.sync_copy(buf, out_hbm.at[pl.ds(base, _ROWS)])


def kernel(x, queue):
    del queue
    return _sc_copy(x)


# speedup vs baseline: 48.1693x; 4.3650x over previous
import jax
import jax.numpy as jnp
from jax.experimental import pallas as pl
from jax.experimental.pallas import tpu as pltpu

BATCH = 16384
FEATURES = 128

_CHUNK_SIZES = (1024, 4096, 5120, 4096, 2048)
_CHUNK_OFFS = tuple(sum(_CHUNK_SIZES[:i]) for i in range(len(_CHUNK_SIZES)))
_NCHUNK = len(_CHUNK_SIZES)


def _copy_body(x_hbm, o_hbm, *rest):
    bufs = rest[:_NCHUNK]
    rsems = rest[_NCHUNK:2 * _NCHUNK]
    wsems = rest[2 * _NCHUNK:]

    def read(k):
        return pltpu.make_async_copy(
            x_hbm.at[pl.ds(_CHUNK_OFFS[k], _CHUNK_SIZES[k])], bufs[k], rsems[k])

    def write(k):
        return pltpu.make_async_copy(
            bufs[k], o_hbm.at[pl.ds(_CHUNK_OFFS[k], _CHUNK_SIZES[k])], wsems[k])

    for k in range(_NCHUNK):
        read(k).start()
    for k in range(_NCHUNK):
        read(k).wait()
        write(k).start()
    for k in range(_NCHUNK):
        write(k).wait()


def kernel(x, queue):
    del queue
    return pl.pallas_call(
        _copy_body,
        in_specs=[pl.BlockSpec(memory_space=pl.ANY)],
        out_specs=pl.BlockSpec(memory_space=pl.ANY),
        out_shape=jax.ShapeDtypeStruct((BATCH, FEATURES), jnp.float32),
        scratch_shapes=(
            [pltpu.VMEM((s, FEATURES), jnp.float32) for s in _CHUNK_SIZES]
            + [pltpu.SemaphoreType.DMA] * (2 * _NCHUNK)
        ),
    )(x)
